# baseline (speedup 1.0000x reference)
import functools

import jax
import jax.numpy as jnp
from jax.experimental import pallas as pl
from jax.experimental.pallas import tpu as pltpu


def _plane_sum_kernel(x_ref, o_ref, *, inv_hw):
    o_ref[...] = (
        jnp.sum(x_ref[...].astype(jnp.float32), axis=0) * inv_hw
    ).astype(o_ref.dtype)


def kernel(x):
    n, c, h, w = x.shape
    hw = h * w

    xt = jnp.transpose(x, (2, 3, 0, 1)).reshape(hw, n, c)

    tn = n
    for cand in (8, 4, 2, 1):
        if n % cand == 0 and hw * cand * c * 4 <= (12 << 20):
            tn = cand
            break
    grid = (n // tn,)

    return pl.pallas_call(
        functools.partial(_plane_sum_kernel, inv_hw=1.0 / hw),
        out_shape=jax.ShapeDtypeStruct((n, c), x.dtype),
        grid=grid,
        in_specs=[pl.BlockSpec((hw, tn, c), lambda i: (0, i, 0))],
        out_specs=pl.BlockSpec((tn, c), lambda i: (i, 0)),
        compiler_params=pltpu.CompilerParams(
            dimension_semantics=("parallel",),
            vmem_limit_bytes=32 << 20,
        ),
        cost_estimate=pl.CostEstimate(
            flops=n * c * hw,
            transcendentals=0,
            bytes_accessed=n * c * hw * 4 + n * c * 4,
        ),
    )(xt)


# speedup vs baseline: 1.1289x; 1.1289x over previous
import functools

import jax
import jax.numpy as jnp
from jax.experimental import pallas as pl
from jax.experimental.pallas import tpu as pltpu


def _plane_sum_kernel(x_ref, o_ref, *, inv_hw):
    o_ref[...] = (
        jnp.sum(x_ref[...].astype(jnp.float32), axis=0) * inv_hw
    ).astype(o_ref.dtype)


def kernel(x):
    n, c, h, w = x.shape
    hw = h * w

    xt = jnp.transpose(x, (2, 3, 0, 1)).reshape(hw, n, c)

    tn = n
    for cand in (32, 16, 8, 4, 2, 1):
        if n % cand == 0 and hw * cand * c * 4 <= (13 << 20):
            tn = cand
            break
    grid = (n // tn,)

    return pl.pallas_call(
        functools.partial(_plane_sum_kernel, inv_hw=1.0 / hw),
        out_shape=jax.ShapeDtypeStruct((n, c), x.dtype),
        grid=grid,
        in_specs=[pl.BlockSpec((hw, tn, c), lambda i: (0, i, 0))],
        out_specs=pl.BlockSpec((tn, c), lambda i: (i, 0)),
        compiler_params=pltpu.CompilerParams(
            dimension_semantics=("parallel",),
            vmem_limit_bytes=32 << 20,
        ),
        cost_estimate=pl.CostEstimate(
            flops=n * c * hw,
            transcendentals=0,
            bytes_accessed=n * c * hw * 4 + n * c * 4,
        ),
    )(xt)


# speedup vs baseline: 1.1814x; 1.0465x over previous
import functools

import jax
import jax.numpy as jnp
from jax.experimental import pallas as pl
from jax.experimental.pallas import tpu as pltpu


def _plane_sum_kernel(x_ref, o_ref, *, inv_hw):
    o_ref[...] = (
        jnp.sum(x_ref[...].astype(jnp.float32), axis=0) * inv_hw
    ).astype(o_ref.dtype)


def kernel(x):
    n, c, h, w = x.shape
    hw = h * w

    xt = jnp.transpose(x, (2, 3, 0, 1)).reshape(hw, n, c)

    tn = n
    for cand in (16, 8, 4, 2, 1):
        if n % cand == 0 and hw * cand * c * 4 <= (12 << 20):
            tn = cand
            break
    grid = (n // tn,)

    return pl.pallas_call(
        functools.partial(_plane_sum_kernel, inv_hw=1.0 / hw),
        out_shape=jax.ShapeDtypeStruct((n, c), x.dtype),
        grid=grid,
        in_specs=[pl.BlockSpec((hw, tn, c), lambda i: (0, i, 0))],
        out_specs=pl.BlockSpec((tn, c), lambda i: (i, 0)),
        compiler_params=pltpu.CompilerParams(
            dimension_semantics=("parallel",),
            vmem_limit_bytes=32 << 20,
        ),
        cost_estimate=pl.CostEstimate(
            flops=n * c * hw,
            transcendentals=0,
            bytes_accessed=n * c * hw * 4 + n * c * 4,
        ),
    )(xt)
